# baseline (speedup 1.0000x reference)
import jax
import jax.numpy as jnp
from jax.experimental import pallas as pl
from jax.experimental.pallas import tpu as pltpu

TOPK = 154
NE = 512
BM = 256
C16 = 65536.0
SIGN32 = -2147483648


def _lane_iota():
    return jax.lax.broadcasted_iota(jnp.int32, (1, 128), 1)


def _perm_mat(j):
    a = jax.lax.broadcasted_iota(jnp.int32, (128, 128), 0)
    b = jax.lax.broadcasted_iota(jnp.int32, (128, 128), 1)
    return ((a ^ j) == b).astype(jnp.float32)


def _self_greater(d1, d2):
    return d1 * C16 + d2 > 0.0


def _cx_within(cols, j, masks):
    pmat = _perm_mat(j)
    lane = _lane_iota()
    bit = (lane & j) != 0
    out = []
    for (w1, w2, ix), tm in zip(cols, masks):
        p1 = jnp.dot(w1, pmat, preferred_element_type=jnp.float32)
        p2 = jnp.dot(w2, pmat, preferred_element_type=jnp.float32)
        pix = jnp.where(bit, pltpu.roll(ix, j, 1), pltpu.roll(ix, 128 - j, 1))
        take_self = _self_greater(w1 - p1, w2 - p2) == tm
        out.append((jnp.where(take_self, w1, p1),
                    jnp.where(take_self, w2, p2),
                    jnp.where(take_self, ix, pix)))
    return out


def _cx_cross(cols, jc, dirs):
    out = list(cols)
    for c in range(len(cols)):
        p = c ^ jc
        if p <= c or p >= len(cols):
            continue
        a, b = cols[c], cols[p]
        gt = _self_greater(a[0] - b[0], a[1] - b[1])
        hi = tuple(jnp.where(gt, ai, bi) for ai, bi in zip(a, b))
        lo = tuple(jnp.where(gt, bi, ai) for ai, bi in zip(a, b))
        out[c], out[p] = (hi, lo) if dirs[c] else (lo, hi)
    return out


def _topk_sort(cols):
    lane = _lane_iota()

    for kp in range(1, 7):
        k = 1 << kp
        j = k >> 1
        while j >= 1:
            tm = ((lane & k) == 0) == ((lane & j) == 0)
            cols = _cx_within(cols, j, [tm] * 4)
            j >>= 1

    for jp in range(6, -1, -1):
        j = 1 << jp
        m_desc = (lane & j) == 0
        m_asc = jnp.logical_not(m_desc)
        cols = _cx_within(cols, j, [m_desc, m_asc, m_desc, m_asc])

    cols = _cx_cross(cols, 1, [True, True, False, False])
    for jp in range(6, -1, -1):
        j = 1 << jp
        m_desc = (lane & j) == 0
        m_asc = jnp.logical_not(m_desc)
        cols = _cx_within(cols, j, [m_desc, m_desc, m_asc, m_asc])

    cols = _cx_cross(cols, 2, [True] * 4)
    cols = cols[:2]
    cols = _cx_cross(cols, 1, [True, True])
    for jp in range(6, -1, -1):
        j = 1 << jp
        m_desc = (lane & j) == 0
        cols = _cx_within(cols, j, [m_desc, m_desc])
    return cols


def _fused_body(x_ref, wt_ref, b_ref, w_ref, i_ref):
    logits = (
        jnp.dot(x_ref[...], wt_ref[...], preferred_element_type=jnp.float32)
        + b_ref[...]
    )

    bits = jax.lax.bitcast_convert_type(logits, jnp.int32)
    s = bits ^ ((bits >> 31) | SIGN32)
    w1_all = jax.lax.shift_right_logical(s, 16)
    w2_all = s & 0xFFFF

    lane = _lane_iota()
    cols = []
    for c in range(NE // 128):
        sl = slice(c * 128, (c + 1) * 128)
        w1 = w1_all[:, sl].astype(jnp.float32)
        w2 = w2_all[:, sl].astype(jnp.float32)
        ix = jnp.broadcast_to((lane + c * 128).astype(jnp.float32),
                              w1.shape)
        cols.append((w1, w2, ix))

    top = _topk_sort(cols)
    w1_t = jnp.concatenate([t[0] for t in top], axis=1)[:, :TOPK]
    w2_t = jnp.concatenate([t[1] for t in top], axis=1)[:, :TOPK]
    ix_t = jnp.concatenate([t[2] for t in top], axis=1)[:, :TOPK]

    s_t = (w1_t.astype(jnp.int32) << 16) | w2_t.astype(jnp.int32)
    vbits = s_t ^ ((jnp.bitwise_not(s_t) >> 31) | SIGN32)
    vals = jax.lax.bitcast_convert_type(vbits, jnp.float32)
    idx = ix_t.astype(jnp.int32)

    e = jnp.exp(vals - vals[:, 0:1])
    w = e / jnp.sum(e, axis=1, keepdims=True)
    w_ref[...] = w
    i_ref[...] = idx


def kernel(x, W, b):
    B, D = x.shape
    assert W.shape[0] == NE and B % BM == 0
    wt = W.T
    b2 = b.reshape(1, NE)
    weights, indices = pl.pallas_call(
        _fused_body,
        grid=(B // BM,),
        in_specs=[
            pl.BlockSpec((BM, D), lambda i: (i, 0)),
            pl.BlockSpec((D, NE), lambda i: (0, 0)),
            pl.BlockSpec((1, NE), lambda i: (0, 0)),
        ],
        out_specs=[
            pl.BlockSpec((BM, TOPK), lambda i: (i, 0)),
            pl.BlockSpec((BM, TOPK), lambda i: (i, 0)),
        ],
        out_shape=[
            jax.ShapeDtypeStruct((B, TOPK), jnp.float32),
            jax.ShapeDtypeStruct((B, TOPK), jnp.int32),
        ],
        compiler_params=pltpu.CompilerParams(
            dimension_semantics=("parallel",),
        ),
    )(x, wt, b2)
    return (weights, indices)


# speedup vs baseline: 1.7145x; 1.7145x over previous
import jax
import jax.numpy as jnp
from jax.experimental import pallas as pl
from jax.experimental.pallas import tpu as pltpu

TOPK = 154
NE = 512
BM = 256


def _lane_iota():
    return jax.lax.broadcasted_iota(jnp.int32, (1, 128), 1)


def _perm_mat(j):
    a = jax.lax.broadcasted_iota(jnp.int32, (128, 128), 0)
    b = jax.lax.broadcasted_iota(jnp.int32, (128, 128), 1)
    return ((a ^ j) == b).astype(jnp.float32)


def _butterfly(v, j):
    pidx = jnp.broadcast_to(_lane_iota() ^ j, v.shape)
    return jnp.take_along_axis(v, pidx, axis=1)


def _cx_within(cols, j, masks):
    pmat = _perm_mat(j)
    out = []
    for (v, ix), tm in zip(cols, masks):
        pv = _butterfly(v, j)
        pix = jnp.dot(ix, pmat, preferred_element_type=jnp.float32)
        take_self = (v > pv) == tm
        out.append((jnp.where(take_self, v, pv),
                    jnp.where(take_self, ix, pix)))
    return out


def _cx_cross(cols, jc, dirs):
    out = list(cols)
    for c in range(len(cols)):
        p = c ^ jc
        if p <= c or p >= len(cols):
            continue
        (av, ai), (bv, bi) = cols[c], cols[p]
        gt = av > bv
        hi = (jnp.where(gt, av, bv), jnp.where(gt, ai, bi))
        lo = (jnp.where(gt, bv, av), jnp.where(gt, bi, ai))
        out[c], out[p] = (hi, lo) if dirs[c] else (lo, hi)
    return out


def _topk_sort(cols):
    lane = _lane_iota()

    for kp in range(1, 7):
        k = 1 << kp
        j = k >> 1
        while j >= 1:
            tm = ((lane & k) == 0) == ((lane & j) == 0)
            cols = _cx_within(cols, j, [tm] * 4)
            j >>= 1

    for jp in range(6, -1, -1):
        j = 1 << jp
        m_desc = (lane & j) == 0
        m_asc = jnp.logical_not(m_desc)
        cols = _cx_within(cols, j, [m_desc, m_asc, m_desc, m_asc])

    cols = _cx_cross(cols, 1, [True, True, False, False])
    for jp in range(6, -1, -1):
        j = 1 << jp
        m_desc = (lane & j) == 0
        m_asc = jnp.logical_not(m_desc)
        cols = _cx_within(cols, j, [m_desc, m_desc, m_asc, m_asc])

    cols = _cx_cross(cols, 2, [True] * 4)
    cols = cols[:2]
    cols = _cx_cross(cols, 1, [True, True])
    for jp in range(6, -1, -1):
        j = 1 << jp
        m_desc = (lane & j) == 0
        cols = _cx_within(cols, j, [m_desc, m_desc])
    return cols


def _fused_body(x_ref, wt_ref, b_ref, w_ref, i_ref):
    logits = (
        jnp.dot(x_ref[...], wt_ref[...], preferred_element_type=jnp.float32)
        + b_ref[...]
    )

    lane = _lane_iota()
    cols = []
    for c in range(NE // 128):
        v = logits[:, c * 128 : (c + 1) * 128]
        ix = jnp.broadcast_to((lane + c * 128).astype(jnp.float32), v.shape)
        cols.append((v, ix))

    top = _topk_sort(cols)
    vals = jnp.concatenate([t[0] for t in top], axis=1)[:, :TOPK]
    idx = jnp.concatenate([t[1] for t in top], axis=1)[:, :TOPK]
    idx = idx.astype(jnp.int32)

    e = jnp.exp(vals - vals[:, 0:1])
    w = e / jnp.sum(e, axis=1, keepdims=True)
    w_ref[...] = w
    i_ref[...] = idx


def kernel(x, W, b):
    B, D = x.shape
    assert W.shape[0] == NE and B % BM == 0
    wt = W.T
    b2 = b.reshape(1, NE)
    weights, indices = pl.pallas_call(
        _fused_body,
        grid=(B // BM,),
        in_specs=[
            pl.BlockSpec((BM, D), lambda i: (i, 0)),
            pl.BlockSpec((D, NE), lambda i: (0, 0)),
            pl.BlockSpec((1, NE), lambda i: (0, 0)),
        ],
        out_specs=[
            pl.BlockSpec((BM, TOPK), lambda i: (i, 0)),
            pl.BlockSpec((BM, TOPK), lambda i: (i, 0)),
        ],
        out_shape=[
            jax.ShapeDtypeStruct((B, TOPK), jnp.float32),
            jax.ShapeDtypeStruct((B, TOPK), jnp.int32),
        ],
        compiler_params=pltpu.CompilerParams(
            dimension_semantics=("parallel",),
        ),
    )(x, wt, b2)
    return (weights, indices)
